# baseline (speedup 1.0000x reference)
import functools

import jax
import jax.numpy as jnp
from jax.experimental import pallas as pl
from jax.experimental.pallas import tpu as pltpu


BT = 1024
NB = 2


def _moe_kernel(x_ref, wr_ref, wg_ref, wu_ref, wd_ref, out_ref,
                wg16, wu16, wd16, gbuf, ubuf, i1s, i2s, w1s, w2s,
                *, eff, n_exp):
    i = pl.program_id(0)

    def _fwd(slot):
        xb = x_ref[...]
        logits = jax.lax.dot_general(xb, wr_ref[...], (((1,), (1,)), ((), ())),
                                     preferred_element_type=jnp.float32)
        i1 = jnp.argmax(logits, axis=-1, keepdims=True)
        v1 = jnp.max(logits, axis=-1, keepdims=True)
        col = jax.lax.broadcasted_iota(jnp.int32, logits.shape, 1)
        masked = jnp.where(col == i1, -jnp.inf, logits)
        i2 = jnp.argmax(masked, axis=-1, keepdims=True)
        v2 = jnp.max(masked, axis=-1, keepdims=True)
        w1 = 1.0 / (1.0 + jnp.exp(v2 - v1))
        i1s[:, slot:slot + 1] = i1.astype(jnp.int32)
        i2s[:, slot:slot + 1] = i2.astype(jnp.int32)
        w1s[:, slot:slot + 1] = w1
        w2s[:, slot:slot + 1] = 1.0 - w1
        xb16 = xb.astype(jnp.bfloat16)
        gbuf[slot] = jnp.dot(xb16, wg16[...],
                             preferred_element_type=jnp.float32).astype(
            jnp.bfloat16)
        ubuf[slot] = jnp.dot(xb16, wu16[...],
                             preferred_element_type=jnp.float32).astype(
            jnp.bfloat16)

    def _bwd(slot):
        g = gbuf[slot].astype(jnp.float32)
        u = ubuf[slot].astype(jnp.float32)
        h = (g * jax.lax.logistic(g)) * u
        e_lane = jax.lax.broadcasted_iota(jnp.int32, h.shape, 1) // eff
        gate = jnp.where(e_lane == i1s[:, slot:slot + 1],
                         w1s[:, slot:slot + 1], 0.0) + jnp.where(
            e_lane == i2s[:, slot:slot + 1], w2s[:, slot:slot + 1], 0.0)
        hg = (h * gate).astype(jnp.bfloat16)
        out_ref[...] = jnp.dot(hg, wd16[...],
                               preferred_element_type=jnp.float32)

    @pl.when(i == 0)
    def _step0():
        wg16[...] = wg_ref[...].astype(jnp.bfloat16).T
        wu16[...] = wu_ref[...].astype(jnp.bfloat16).T
        _fwd(0)

    @pl.when(i == 1)
    def _prep_wd():
        for e in range(n_exp):
            wd16[e * eff:(e + 1) * eff, :] = wd_ref[e].astype(jnp.bfloat16).T

    for step in range(1, NB):
        @pl.when(i == step)
        def _mid(step=step):
            _fwd(step % 2)
            _bwd((step - 1) % 2)

    @pl.when(i == NB)
    def _last():
        _bwd((NB - 1) % 2)


def kernel(x, base_gate_w, base_up_w, base_down_w, router_weight,
           expert_gate_w, expert_up_w, expert_down_w):
    batch, seq_len, hidden = x.shape
    n_exp, eff, _ = expert_gate_w.shape
    t = batch * seq_len
    x2d = x.reshape(t, hidden)

    wg = expert_gate_w.reshape(n_exp * eff, hidden)
    wu = expert_up_w.reshape(n_exp * eff, hidden)
    wd = expert_down_w

    assert t // BT == NB
    grid = (NB + 1,)
    out = pl.pallas_call(
        functools.partial(_moe_kernel, eff=eff, n_exp=n_exp),
        grid=grid,
        in_specs=[
            pl.BlockSpec((BT, hidden), lambda i: (i - i // NB, 0)),
            pl.BlockSpec((n_exp, hidden), lambda i: (0, 0)),
            pl.BlockSpec((n_exp * eff, hidden), lambda i: (0, 0)),
            pl.BlockSpec((n_exp * eff, hidden), lambda i: (0, 0)),
            pl.BlockSpec((n_exp, hidden, eff), lambda i: (0, 0, 0)),
        ],
        out_specs=pl.BlockSpec(
            (BT, hidden), lambda i: ((i * NB) // (NB + 1), 0)),
        out_shape=jax.ShapeDtypeStruct((t, hidden), jnp.float32),
        scratch_shapes=[
            pltpu.VMEM((hidden, n_exp * eff), jnp.bfloat16),
            pltpu.VMEM((hidden, n_exp * eff), jnp.bfloat16),
            pltpu.VMEM((n_exp * eff, hidden), jnp.bfloat16),
            pltpu.VMEM((2, BT, n_exp * eff), jnp.bfloat16),
            pltpu.VMEM((2, BT, n_exp * eff), jnp.bfloat16),
            pltpu.VMEM((BT, 2), jnp.int32),
            pltpu.VMEM((BT, 2), jnp.int32),
            pltpu.VMEM((BT, 2), jnp.float32),
            pltpu.VMEM((BT, 2), jnp.float32),
        ],
        compiler_params=pltpu.CompilerParams(
            vmem_limit_bytes=66981888,
        ),
    )(x2d, router_weight, wg, wu, wd)

    return out.astype(x.dtype).reshape(batch, seq_len, hidden)


# speedup vs baseline: 1.0599x; 1.0599x over previous
import functools

import jax
import jax.numpy as jnp
from jax.experimental import pallas as pl
from jax.experimental.pallas import tpu as pltpu


BT = 512
NB = 4


def _moe_kernel(x_ref, wr_ref, wg_ref, wu_ref, wd_ref, out_ref,
                wg16, wu16, wd16, gbuf, ubuf, i1s, i2s, w1s, w2s,
                *, eff, n_exp):
    i = pl.program_id(0)

    def _fwd(slot):
        xb = x_ref[...]
        logits = jax.lax.dot_general(xb, wr_ref[...], (((1,), (1,)), ((), ())),
                                     preferred_element_type=jnp.float32)
        i1 = jnp.argmax(logits, axis=-1, keepdims=True)
        v1 = jnp.max(logits, axis=-1, keepdims=True)
        col = jax.lax.broadcasted_iota(jnp.int32, logits.shape, 1)
        masked = jnp.where(col == i1, -jnp.inf, logits)
        i2 = jnp.argmax(masked, axis=-1, keepdims=True)
        v2 = jnp.max(masked, axis=-1, keepdims=True)
        w1 = 1.0 / (1.0 + jnp.exp(v2 - v1))
        i1s[:, slot:slot + 1] = i1.astype(jnp.int32)
        i2s[:, slot:slot + 1] = i2.astype(jnp.int32)
        w1s[:, slot:slot + 1] = w1
        w2s[:, slot:slot + 1] = 1.0 - w1
        xb16 = xb.astype(jnp.bfloat16)
        gbuf[slot] = jnp.dot(xb16, wg16[...],
                             preferred_element_type=jnp.float32)
        ubuf[slot] = jnp.dot(xb16, wu16[...],
                             preferred_element_type=jnp.float32)

    def _bwd(slot):
        g = gbuf[slot]
        u = ubuf[slot]
        h = (g * jax.lax.logistic(g)) * u
        e_lane = jax.lax.broadcasted_iota(jnp.int32, h.shape, 1) // eff
        gate = jnp.where(e_lane == i1s[:, slot:slot + 1],
                         w1s[:, slot:slot + 1], 0.0) + jnp.where(
            e_lane == i2s[:, slot:slot + 1], w2s[:, slot:slot + 1], 0.0)
        hg = (h * gate).astype(jnp.bfloat16)
        out_ref[...] = jnp.dot(hg, wd16[...],
                               preferred_element_type=jnp.float32)

    @pl.when(i == 0)
    def _step0():
        wg16[...] = wg_ref[...].astype(jnp.bfloat16).T
        wu16[...] = wu_ref[...].astype(jnp.bfloat16).T
        _fwd(0)

    @pl.when(i == 1)
    def _prep_wd():
        for e in range(n_exp):
            wd16[e * eff:(e + 1) * eff, :] = wd_ref[e].astype(jnp.bfloat16).T

    for step in range(1, NB):
        @pl.when(i == step)
        def _mid(step=step):
            _fwd(step % 2)
            _bwd((step - 1) % 2)

    @pl.when(i == NB)
    def _last():
        _bwd((NB - 1) % 2)


def kernel(x, base_gate_w, base_up_w, base_down_w, router_weight,
           expert_gate_w, expert_up_w, expert_down_w):
    batch, seq_len, hidden = x.shape
    n_exp, eff, _ = expert_gate_w.shape
    t = batch * seq_len
    x2d = x.reshape(t, hidden)

    wg = expert_gate_w.reshape(n_exp * eff, hidden)
    wu = expert_up_w.reshape(n_exp * eff, hidden)
    wd = expert_down_w

    assert t // BT == NB
    grid = (NB + 1,)
    out = pl.pallas_call(
        functools.partial(_moe_kernel, eff=eff, n_exp=n_exp),
        grid=grid,
        in_specs=[
            pl.BlockSpec((BT, hidden), lambda i: (i - i // NB, 0)),
            pl.BlockSpec((n_exp, hidden), lambda i: (0, 0)),
            pl.BlockSpec((n_exp * eff, hidden), lambda i: (0, 0)),
            pl.BlockSpec((n_exp * eff, hidden), lambda i: (0, 0)),
            pl.BlockSpec((n_exp, hidden, eff), lambda i: (0, 0, 0)),
        ],
        out_specs=pl.BlockSpec(
            (BT, hidden), lambda i: ((i * NB) // (NB + 1), 0)),
        out_shape=jax.ShapeDtypeStruct((t, hidden), jnp.float32),
        scratch_shapes=[
            pltpu.VMEM((hidden, n_exp * eff), jnp.bfloat16),
            pltpu.VMEM((hidden, n_exp * eff), jnp.bfloat16),
            pltpu.VMEM((n_exp * eff, hidden), jnp.bfloat16),
            pltpu.VMEM((2, BT, n_exp * eff), jnp.float32),
            pltpu.VMEM((2, BT, n_exp * eff), jnp.float32),
            pltpu.VMEM((BT, 2), jnp.int32),
            pltpu.VMEM((BT, 2), jnp.int32),
            pltpu.VMEM((BT, 2), jnp.float32),
            pltpu.VMEM((BT, 2), jnp.float32),
        ],
        compiler_params=pltpu.CompilerParams(
            vmem_limit_bytes=66981888,
        ),
    )(x2d, router_weight, wg, wu, wd)

    return out.astype(x.dtype).reshape(batch, seq_len, hidden)


# speedup vs baseline: 1.0630x; 1.0030x over previous
import functools

import jax
import jax.numpy as jnp
from jax.experimental import pallas as pl
from jax.experimental.pallas import tpu as pltpu


BT = 512
NB = 4


def _moe_kernel(x_ref, wr_ref, wg_ref, wu_ref, wd_ref, out_ref,
                wg16, wu16, wd16, gbuf, ubuf, i1s, i2s, w1s, w2s,
                *, eff, n_exp):
    i = pl.program_id(0)

    def _fwd(slot):
        xb = x_ref[...]
        logits = jax.lax.dot_general(xb, wr_ref[...], (((1,), (1,)), ((), ())),
                                     preferred_element_type=jnp.float32)
        i1 = jnp.argmax(logits, axis=-1, keepdims=True)
        v1 = jnp.max(logits, axis=-1, keepdims=True)
        col = jax.lax.broadcasted_iota(jnp.int32, logits.shape, 1)
        masked = jnp.where(col == i1, -jnp.inf, logits)
        i2 = jnp.argmax(masked, axis=-1, keepdims=True)
        v2 = jnp.max(masked, axis=-1, keepdims=True)
        w1 = 1.0 / (1.0 + jnp.exp(v2 - v1))
        i1s[:, slot:slot + 1] = i1.astype(jnp.int32)
        i2s[:, slot:slot + 1] = i2.astype(jnp.int32)
        w1s[:, slot:slot + 1] = w1
        w2s[:, slot:slot + 1] = 1.0 - w1
        xb16 = xb.astype(jnp.bfloat16)
        gbuf[slot] = jnp.dot(xb16, wg16[...],
                             preferred_element_type=jnp.float32)
        ubuf[slot] = jnp.dot(xb16, wu16[...],
                             preferred_element_type=jnp.float32)

    def _bwd(slot):
        g = gbuf[slot]
        u = ubuf[slot]
        h = (g * jax.lax.logistic(g)) * u
        e_lane = jax.lax.broadcasted_iota(jnp.int32, h.shape, 1) // eff
        gate = jnp.where(e_lane == i1s[:, slot:slot + 1],
                         w1s[:, slot:slot + 1], 0.0) + jnp.where(
            e_lane == i2s[:, slot:slot + 1], w2s[:, slot:slot + 1], 0.0)
        hg = (h * gate).astype(jnp.bfloat16)
        out_ref[...] = jnp.dot(hg, wd16[...],
                               preferred_element_type=jnp.float32)

    @pl.when(i == 0)
    def _step0():
        wg16[...] = wg_ref[...].astype(jnp.bfloat16).T
        wu16[...] = wu_ref[...].astype(jnp.bfloat16).T
        for e in range(n_exp // 2):
            wd16[e * eff:(e + 1) * eff, :] = wd_ref[0, e].astype(jnp.bfloat16).T
        _fwd(0)

    @pl.when(i == 1)
    def _prep_wd():
        for e in range(n_exp // 2):
            wd16[(n_exp // 2 + e) * eff:(n_exp // 2 + e + 1) * eff, :] = (
                wd_ref[0, e].astype(jnp.bfloat16).T)

    for step in range(1, NB):
        @pl.when(i == step)
        def _mid(step=step):
            _fwd(step % 2)
            _bwd((step - 1) % 2)

    @pl.when(i == NB)
    def _last():
        _bwd((NB - 1) % 2)


def kernel(x, base_gate_w, base_up_w, base_down_w, router_weight,
           expert_gate_w, expert_up_w, expert_down_w):
    batch, seq_len, hidden = x.shape
    n_exp, eff, _ = expert_gate_w.shape
    t = batch * seq_len
    x2d = x.reshape(t, hidden)

    wg = expert_gate_w.reshape(n_exp * eff, hidden)
    wu = expert_up_w.reshape(n_exp * eff, hidden)
    wd = expert_down_w.reshape(2, n_exp // 2, hidden, eff)

    assert t // BT == NB
    grid = (NB + 1,)
    out = pl.pallas_call(
        functools.partial(_moe_kernel, eff=eff, n_exp=n_exp),
        grid=grid,
        in_specs=[
            pl.BlockSpec((BT, hidden), lambda i: (i - i // NB, 0)),
            pl.BlockSpec((n_exp, hidden), lambda i: (0, 0)),
            pl.BlockSpec((n_exp * eff, hidden), lambda i: (0, 0)),
            pl.BlockSpec((n_exp * eff, hidden), lambda i: (0, 0)),
            pl.BlockSpec((1, n_exp // 2, hidden, eff),
                         lambda i: ((i + NB) // (NB + 1), 0, 0, 0)),
        ],
        out_specs=pl.BlockSpec(
            (BT, hidden), lambda i: ((i * NB) // (NB + 1), 0)),
        out_shape=jax.ShapeDtypeStruct((t, hidden), jnp.float32),
        scratch_shapes=[
            pltpu.VMEM((hidden, n_exp * eff), jnp.bfloat16),
            pltpu.VMEM((hidden, n_exp * eff), jnp.bfloat16),
            pltpu.VMEM((n_exp * eff, hidden), jnp.bfloat16),
            pltpu.VMEM((2, BT, n_exp * eff), jnp.float32),
            pltpu.VMEM((2, BT, n_exp * eff), jnp.float32),
            pltpu.VMEM((BT, 2), jnp.int32),
            pltpu.VMEM((BT, 2), jnp.int32),
            pltpu.VMEM((BT, 2), jnp.float32),
            pltpu.VMEM((BT, 2), jnp.float32),
        ],
        compiler_params=pltpu.CompilerParams(
            vmem_limit_bytes=66981888,
        ),
    )(x2d, router_weight, wg, wu, wd)

    return out.astype(x.dtype).reshape(batch, seq_len, hidden)
